# Initial kernel scaffold; baseline (speedup 1.0000x reference)
import jax
import jax.numpy as jnp
from jax.experimental import pallas as pl


def kernel(x, edge_index, edge_attr, batch, W1, att_src1, att_dst1, We1, att_e1, b1, W2, att_src2, att_dst2, We2, att_e2, b2):
    raise NotImplementedError("write your pallas kernel here")


# speedup vs baseline: 13.8847x; 13.8847x over previous
import functools

import jax
import jax.numpy as jnp
from jax import lax
from jax.experimental import pallas as pl
from jax.experimental.pallas import tpu as pltpu
from jax.experimental.pallas import tpu_sc as plsc

N = 10000
E = 320000
G = 64
NC, NS = 2, 16
EPS = E // NS
HALF = N // NC
TRASH = HALF
ACC_ROWS = HALF + 8

ROW1, P1, CHUNK1 = 128, 96, 80
ROW2, P2, CHUNK2 = 256, 136, 80
HOFF = 16
ZROWS = 80
SUB_OFF = 312
SUB_ROWS = 320


def _leaky(t):
    return jnp.where(t > 0, t, 0.2 * t)


def _make_edge_pass(row_w, d_w, p_w, chunk, nmsg, ch, dcol, av_off):
    n_iter = EPS // chunk
    mesh = plsc.VectorSubcoreMesh(core_axis_name="c", subcore_axis_name="s")

    @functools.partial(
        pl.kernel,
        mesh=mesh,
        compiler_params=pltpu.CompilerParams(use_tc_tiling_on_sc=False),
        out_type=jax.ShapeDtypeStruct((N, p_w), jnp.float32),
        scratch_types=[
            pltpu.VMEM((chunk,), jnp.int32),
            pltpu.VMEM((chunk,), jnp.int32),
            pltpu.VMEM((chunk, row_w), jnp.float32),
            pltpu.VMEM((chunk, d_w), jnp.float32),
            pltpu.VMEM((chunk, 128), jnp.float32),
            pltpu.VMEM((chunk, p_w), jnp.float32),
            pltpu.VMEM((ZROWS, p_w), jnp.float32),
            pltpu.VMEM_SHARED((ACC_ROWS, p_w), jnp.float32),
            pltpu.SemaphoreType.DMA,
            pltpu.SemaphoreType.DMA,
        ],
    )
    def edge_pass(tsrc, adstt, aet, srcix, dstix, out,
                  sidx, didx, srows, drows, aebuf, payload, zbuf, acc,
                  sem, sem2):
        c = lax.axis_index("c")
        s = lax.axis_index("s")
        iota = lax.iota(jnp.int32, 16)
        zero = (iota * jnp.int32(0)).astype(jnp.float32)
        coff = c * jnp.int32(HALF)
        lohalf = iota < jnp.int32(8)
        one8 = jnp.where(iota == jnp.int32(8), 1.0, 0.0).astype(jnp.float32)

        def zrow(r, carry):
            for k in range(p_w // 16):
                zbuf[r, pl.ds(k * 16, 16)] = zero
            return carry

        lax.fori_loop(0, ZROWS, zrow, 0)

        def zcopy(i, carry):
            pltpu.sync_copy(zbuf, acc.at[pl.ds(s * SUB_OFF + i * ZROWS, ZROWS)])
            return carry

        lax.fori_loop(0, SUB_ROWS // ZROWS, zcopy, 0)
        plsc.subcore_barrier()

        def chunk_body(i, carry):
            base = s * EPS + i * chunk
            pltpu.sync_copy(srcix.at[pl.ds(base, chunk)], sidx)
            pltpu.sync_copy(dstix.at[pl.ds(base, chunk)], didx)
            cp1 = pltpu.async_copy(tsrc.at[sidx], srows, sem)
            cp2 = pltpu.async_copy(adstt.at[didx], drows, sem2)
            pltpu.sync_copy(aet.at[pl.ds(base, chunk)], aebuf)
            cp2.wait()
            cp1.wait()

            def fix_idx(v, fcarry):
                d = didx[pl.ds(v * 16, 16)] - coff
                bad = (d < 0) | (d >= HALF)
                didx[pl.ds(v * 16, 16)] = jnp.where(bad, jnp.int32(TRASH), d)
                return fcarry

            lax.fori_loop(0, chunk // 16, fix_idx, 0)

            def edge_body(e, ecarry):
                sv = srows[e, pl.ds(0, 16)]
                dv = drows[e, pl.ds(dcol, 16)]
                av = aebuf[e, pl.ds(av_off, 16)]
                w = jnp.exp(_leaky(sv + dv + av))
                if ch == 8:
                    payload[e, pl.ds(0, 16)] = av
                    payload[e, pl.ds(16, 16)] = w
                    for j in range(nmsg):
                        bc = jnp.where(lohalf, w[1 + 2 * j], w[2 + 2 * j])
                        hv = srows[e, pl.ds(HOFF + j * 16, 16)]
                        payload[e, pl.ds(32 + j * 16, 16)] = bc * hv
                else:
                    w2 = w[1]
                    for j in range(nmsg):
                        hv = srows[e, pl.ds(HOFF + j * 16, 16)]
                        payload[e, pl.ds(j * 16, 16)] = hv * w2
                    tail = srows[e, pl.ds(HOFF + 120, 16)] + one8
                    payload[e, pl.ds(120, 16)] = tail * w2
                return ecarry

            lax.fori_loop(0, chunk, edge_body, 0)
            pltpu.sync_copy(payload, acc.at[didx], add=True)
            return carry

        lax.fori_loop(0, n_iter, chunk_body, 0)
        plsc.subcore_barrier()
        pltpu.sync_copy(acc.at[pl.ds(s * SUB_OFF, SUB_ROWS)],
                        out.at[pl.ds(c * HALF + s * SUB_OFF, SUB_ROWS)])

    return edge_pass


_edge_pass1 = _make_edge_pass(ROW1, ROW1, P1, CHUNK1, nmsg=4, ch=8,
                              dcol=112, av_off=0)
_edge_pass2 = _make_edge_pass(ROW2, 128, P2, CHUNK2, nmsg=8, ch=128,
                              dcol=0, av_off=16)


def _node1(x, W1, as1, ad1):
    BN = 1000
    NB = N // BN

    def body(x_ref, w_ref, as_ref, ad_ref, ts_ref):
        h = jnp.dot(x_ref[...], w_ref[...], preferred_element_type=jnp.float32)
        hr = h.reshape(BN, 8, 8)
        asv = (hr * as_ref[...][None]).sum(-1)
        adv = (hr * ad_ref[...][None]).sum(-1)
        z1 = jnp.zeros((BN, 1), jnp.float32)
        z7 = jnp.zeros((BN, 7), jnp.float32)
        z32 = jnp.zeros((BN, 32), jnp.float32)
        ts_ref[...] = jnp.concatenate([z1, asv, z7, h, z32, z1, adv, z7],
                                      axis=1)

    return pl.pallas_call(
        body,
        grid=(NB,),
        in_specs=[pl.BlockSpec((BN, 128), lambda i: (i, 0)),
                  pl.BlockSpec((128, 64), lambda i: (0, 0)),
                  pl.BlockSpec((8, 8), lambda i: (0, 0)),
                  pl.BlockSpec((8, 8), lambda i: (0, 0))],
        out_specs=pl.BlockSpec((BN, ROW1), lambda i: (i, 0)),
        out_shape=jax.ShapeDtypeStruct((N, ROW1), jnp.float32),
    )(x, W1, as1, ad1)


def _ae_tab(edge_attr, We1, ae1, We2, ae2):
    BB = 2000

    def body(ea_ref, we1_ref, a1_ref, we2_ref, a2_ref, out_ref):
        wef1 = (we1_ref[...].reshape(16, 8, 8) * a1_ref[...][None]).sum(-1)
        wef2 = jnp.dot(we2_ref[...], a2_ref[...],
                       preferred_element_type=jnp.float32)
        ea = ea_ref[...]
        t1 = jnp.dot(ea, wef1, preferred_element_type=jnp.float32)
        t2 = jnp.dot(ea, wef2, preferred_element_type=jnp.float32)
        one = jnp.ones((BB, 1), jnp.float32)
        z1 = jnp.zeros((BB, 1), jnp.float32)
        z6 = jnp.zeros((BB, 6), jnp.float32)
        z14 = jnp.zeros((BB, 14), jnp.float32)
        z96 = jnp.zeros((BB, 96), jnp.float32)
        out_ref[...] = jnp.concatenate([one, t1, t2, z6, z1, t2, z14, z96],
                                       axis=1)

    return pl.pallas_call(
        body,
        grid=(E // BB,),
        in_specs=[pl.BlockSpec((BB, 16), lambda i: (i, 0)),
                  pl.BlockSpec((16, 64), lambda i: (0, 0)),
                  pl.BlockSpec((8, 8), lambda i: (0, 0)),
                  pl.BlockSpec((16, 128), lambda i: (0, 0)),
                  pl.BlockSpec((128, 1), lambda i: (0, 0))],
        out_specs=pl.BlockSpec((BB, 128), lambda i: (i, 0)),
        out_shape=jax.ShapeDtypeStruct((E, 128), jnp.float32),
    )(edge_attr, We1, ae1, We2, ae2)


def _mid(acc1, ts1, W2, as2, ad2, b1):
    BN = 1000
    NB = N // BN
    hb = N // (BN * NC)

    def body(acc_ref, ts1_ref, w2_ref, as2_ref, ad2_ref, b1_ref,
             ts2_ref, ad2t_ref, aux_ref):
        a = acc_ref[...]
        deg = jnp.maximum(a[:, 0:1], 1.0)
        ae1l = a[:, 1:9] / deg
        ae2l = a[:, 9:10] / deg
        ts1 = ts1_ref[...]
        asrc1 = ts1[:, 1:9]
        h1 = ts1[:, 16:80]
        adst1 = ts1[:, 113:121]
        wl = jnp.exp(_leaky(asrc1 + adst1 + ae1l))
        denom = a[:, 17:25] + wl + 1e-16
        hr = h1.reshape(BN, 8, 8)
        msg = a[:, 32:96].reshape(BN, 8, 8) + wl[:, :, None] * hr
        o1 = (msg / denom[:, :, None]).reshape(BN, 64) + b1_ref[...]
        hact = jnp.where(o1 > 0, o1, jnp.exp(o1) - 1.0)
        h2 = jnp.dot(hact, w2_ref[...], preferred_element_type=jnp.float32)
        asrc2 = jnp.dot(h2, as2_ref[...], preferred_element_type=jnp.float32)
        adst2 = jnp.dot(h2, ad2_ref[...], preferred_element_type=jnp.float32)
        z1 = jnp.zeros((BN, 1), jnp.float32)
        z14 = jnp.zeros((BN, 14), jnp.float32)
        z112 = jnp.zeros((BN, 112), jnp.float32)
        z126 = jnp.zeros((BN, 126), jnp.float32)
        ts2_ref[...] = jnp.concatenate([z1, asrc2, z14, h2, z112], axis=1)
        ad2t_ref[...] = jnp.concatenate([z1, adst2, z126], axis=1)
        wl2 = jnp.exp(_leaky(asrc2 + adst2 + ae2l))
        aux_ref[...] = jnp.concatenate([wl2, jnp.zeros((BN, 7), jnp.float32)],
                                       axis=1)

    return pl.pallas_call(
        body,
        grid=(NB,),
        in_specs=[pl.BlockSpec((BN, P1), lambda i: (i, 0)),
                  pl.BlockSpec((BN, ROW1), lambda i: (i, 0)),
                  pl.BlockSpec((64, 128), lambda i: (0, 0)),
                  pl.BlockSpec((128, 1), lambda i: (0, 0)),
                  pl.BlockSpec((128, 1), lambda i: (0, 0)),
                  pl.BlockSpec((1, 64), lambda i: (0, 0))],
        out_specs=(pl.BlockSpec((BN, ROW2), lambda i: (i, 0)),
                   pl.BlockSpec((BN, 128), lambda i: (i, 0)),
                   pl.BlockSpec((BN, 8), lambda i: (i, 0))),
        out_shape=(jax.ShapeDtypeStruct((N, ROW2), jnp.float32),
                   jax.ShapeDtypeStruct((N, 128), jnp.float32),
                   jax.ShapeDtypeStruct((N, 8), jnp.float32)),
    )(acc1, ts1, W2, as2, ad2, b1)


def _final(acc2, ts2, aux, b2, batch3d):
    BN = 1000
    NB = N // BN
    hb = N // (BN * NC)

    def body(acc_ref, ts2_ref, aux_ref, b2_ref, bt_ref, out_ref, cnt_ref):
        i = pl.program_id(0)
        a = acc_ref[...]
        wl2 = aux_ref[...][:, 0:1]
        h2 = ts2_ref[...][:, 16:144]
        denom = a[:, 128:129] + wl2 + 1e-16
        o2 = (a[:, 0:128] + wl2 * h2) / denom + b2_ref[...]
        gids = lax.broadcasted_iota(jnp.int32, (G, BN), 0)
        mask = (bt_ref[0] == gids).astype(jnp.float32)
        ssum = jnp.dot(mask, o2, preferred_element_type=jnp.float32)
        cb = jnp.concatenate(
            [mask.sum(axis=1, keepdims=True), jnp.zeros((G, 7), jnp.float32)],
            axis=1)

        @pl.when(i == 0)
        def _():
            out_ref[...] = ssum
            cnt_ref[...] = cb

        @pl.when(i > 0)
        def _():
            out_ref[...] += ssum
            cnt_ref[...] += cb

        @pl.when(i == NB - 1)
        def _():
            out_ref[...] = out_ref[...] / jnp.maximum(cnt_ref[...][:, 0:1], 1.0)

    out, _cnt = pl.pallas_call(
        body,
        grid=(NB,),
        in_specs=[pl.BlockSpec((BN, P2), lambda i: (i, 0)),
                  pl.BlockSpec((BN, ROW2), lambda i: (i, 0)),
                  pl.BlockSpec((BN, 8), lambda i: (i, 0)),
                  pl.BlockSpec((1, 128), lambda i: (0, 0)),
                  pl.BlockSpec((1, 1, BN), lambda i: (i, 0, 0))],
        out_specs=(pl.BlockSpec((G, 128), lambda i: (0, 0)),
                   pl.BlockSpec((G, 8), lambda i: (0, 0))),
        out_shape=(jax.ShapeDtypeStruct((G, 128), jnp.float32),
                   jax.ShapeDtypeStruct((G, 8), jnp.float32)),
    )(acc2, ts2, aux, b2, batch3d)
    return out


def kernel(x, edge_index, edge_attr, batch,
           W1, att_src1, att_dst1, We1, att_e1, b1,
           W2, att_src2, att_dst2, We2, att_e2, b2):
    src = edge_index[0]
    dst = edge_index[1]
    ts1 = _node1(x, W1, att_src1.reshape(8, 8), att_dst1.reshape(8, 8))
    aet = _ae_tab(edge_attr, We1, att_e1.reshape(8, 8), We2,
                  att_e2.reshape(128, 1))
    acc1 = _edge_pass1(ts1, ts1, aet, src, dst)
    ts2, ad2t, aux = _mid(acc1, ts1, W2,
                          att_src2.reshape(128, 1), att_dst2.reshape(128, 1),
                          b1.reshape(1, 64))
    acc2 = _edge_pass2(ts2, ad2t, aet, src, dst)
    return _final(acc2, ts2, aux, b2.reshape(1, 128), batch.reshape(10, 1, 1000))


# speedup vs baseline: 14.4645x; 1.0418x over previous
import functools

import jax
import jax.numpy as jnp
from jax import lax
from jax.experimental import pallas as pl
from jax.experimental.pallas import tpu as pltpu
from jax.experimental.pallas import tpu_sc as plsc

N = 10000
E = 320000
G = 64
NC, NS = 2, 16
EPS = E // NS
HALF = N // NC
TRASH = HALF
ACC_ROWS = HALF + 8

ROW1, P1, CHUNK1 = 128, 96, 160
ROW2, P2, CHUNK2 = 256, 136, 80
HOFF = 16
ZROWS = 80
SUB_OFF = 312
SUB_ROWS = 320


def _leaky(t):
    return jnp.where(t > 0, t, 0.2 * t)


def _make_edge_pass(row_w, d_w, p_w, chunk, nmsg, ch, dcol, av_off):
    n_iter = EPS // chunk
    mesh = plsc.VectorSubcoreMesh(core_axis_name="c", subcore_axis_name="s")

    @functools.partial(
        pl.kernel,
        mesh=mesh,
        compiler_params=pltpu.CompilerParams(use_tc_tiling_on_sc=False),
        out_type=jax.ShapeDtypeStruct((N, p_w), jnp.float32),
        scratch_types=[
            pltpu.VMEM((chunk,), jnp.int32),
            pltpu.VMEM((chunk,), jnp.int32),
            pltpu.VMEM((chunk, row_w), jnp.float32),
            pltpu.VMEM((chunk, d_w), jnp.float32),
            pltpu.VMEM((chunk, 128), jnp.float32),
            pltpu.VMEM((chunk, p_w), jnp.float32),
            pltpu.VMEM((ZROWS, p_w), jnp.float32),
            pltpu.VMEM_SHARED((ACC_ROWS, p_w), jnp.float32),
            pltpu.SemaphoreType.DMA,
            pltpu.SemaphoreType.DMA,
        ],
    )
    def edge_pass(tsrc, adstt, aet, srcix, dstix, out,
                  sidx, didx, srows, drows, aebuf, payload, zbuf, acc,
                  sem, sem2):
        c = lax.axis_index("c")
        s = lax.axis_index("s")
        iota = lax.iota(jnp.int32, 16)
        zero = (iota * jnp.int32(0)).astype(jnp.float32)
        coff = c * jnp.int32(HALF)
        lohalf = iota < jnp.int32(8)
        one8 = jnp.where(iota == jnp.int32(8), 1.0, 0.0).astype(jnp.float32)

        def zrow(r, carry):
            for k in range(p_w // 16):
                zbuf[r, pl.ds(k * 16, 16)] = zero
            return carry

        lax.fori_loop(0, ZROWS, zrow, 0)

        def zcopy(i, carry):
            pltpu.sync_copy(zbuf, acc.at[pl.ds(s * SUB_OFF + i * ZROWS, ZROWS)])
            return carry

        lax.fori_loop(0, SUB_ROWS // ZROWS, zcopy, 0)
        plsc.subcore_barrier()

        def chunk_body(i, carry):
            base = s * EPS + i * chunk
            pltpu.sync_copy(srcix.at[pl.ds(base, chunk)], sidx)
            pltpu.sync_copy(dstix.at[pl.ds(base, chunk)], didx)
            cp1 = pltpu.async_copy(tsrc.at[sidx], srows, sem)
            cp2 = pltpu.async_copy(adstt.at[didx], drows, sem2)
            pltpu.sync_copy(aet.at[pl.ds(base, chunk)], aebuf)
            cp2.wait()
            cp1.wait()

            def fix_idx(v, fcarry):
                d = didx[pl.ds(v * 16, 16)] - coff
                bad = (d < 0) | (d >= HALF)
                didx[pl.ds(v * 16, 16)] = jnp.where(bad, jnp.int32(TRASH), d)
                return fcarry

            lax.fori_loop(0, chunk // 16, fix_idx, 0)

            def edge_body(e, ecarry):
                sv = srows[e, pl.ds(0, 16)]
                dv = drows[e, pl.ds(dcol, 16)]
                av = aebuf[e, pl.ds(av_off, 16)]
                w = jnp.exp(_leaky(sv + dv + av))
                if ch == 8:
                    payload[e, pl.ds(0, 16)] = av
                    payload[e, pl.ds(16, 16)] = w
                    for j in range(nmsg):
                        bc = jnp.where(lohalf, w[1 + 2 * j], w[2 + 2 * j])
                        hv = srows[e, pl.ds(HOFF + j * 16, 16)]
                        payload[e, pl.ds(32 + j * 16, 16)] = bc * hv
                else:
                    w2 = w[1]
                    for j in range(nmsg):
                        hv = srows[e, pl.ds(HOFF + j * 16, 16)]
                        payload[e, pl.ds(j * 16, 16)] = hv * w2
                    tail = srows[e, pl.ds(HOFF + 120, 16)] + one8
                    payload[e, pl.ds(120, 16)] = tail * w2
                return ecarry

            lax.fori_loop(0, chunk, edge_body, 0)
            pltpu.sync_copy(payload, acc.at[didx], add=True)
            return carry

        lax.fori_loop(0, n_iter, chunk_body, 0)
        plsc.subcore_barrier()
        pltpu.sync_copy(acc.at[pl.ds(s * SUB_OFF, SUB_ROWS)],
                        out.at[pl.ds(c * HALF + s * SUB_OFF, SUB_ROWS)])

    return edge_pass


_edge_pass1 = _make_edge_pass(ROW1, ROW1, P1, CHUNK1, nmsg=4, ch=8,
                              dcol=112, av_off=0)
_edge_pass2 = _make_edge_pass(ROW2, 128, P2, CHUNK2, nmsg=8, ch=128,
                              dcol=0, av_off=16)


def _node1(x, W1, as1, ad1):
    BN = 1000
    NB = N // BN

    def body(x_ref, w_ref, as_ref, ad_ref, ts_ref):
        h = jnp.dot(x_ref[...], w_ref[...], preferred_element_type=jnp.float32)
        hr = h.reshape(BN, 8, 8)
        asv = (hr * as_ref[...][None]).sum(-1)
        adv = (hr * ad_ref[...][None]).sum(-1)
        z1 = jnp.zeros((BN, 1), jnp.float32)
        z7 = jnp.zeros((BN, 7), jnp.float32)
        z32 = jnp.zeros((BN, 32), jnp.float32)
        ts_ref[...] = jnp.concatenate([z1, asv, z7, h, z32, z1, adv, z7],
                                      axis=1)

    return pl.pallas_call(
        body,
        grid=(NB,),
        in_specs=[pl.BlockSpec((BN, 128), lambda i: (i, 0)),
                  pl.BlockSpec((128, 64), lambda i: (0, 0)),
                  pl.BlockSpec((8, 8), lambda i: (0, 0)),
                  pl.BlockSpec((8, 8), lambda i: (0, 0))],
        out_specs=pl.BlockSpec((BN, ROW1), lambda i: (i, 0)),
        out_shape=jax.ShapeDtypeStruct((N, ROW1), jnp.float32),
    )(x, W1, as1, ad1)


def _ae_tab(edge_attr, We1, ae1, We2, ae2):
    BB = 2000

    def body(ea_ref, we1_ref, a1_ref, we2_ref, a2_ref, out_ref):
        wef1 = (we1_ref[...].reshape(16, 8, 8) * a1_ref[...][None]).sum(-1)
        wef2 = jnp.dot(we2_ref[...], a2_ref[...],
                       preferred_element_type=jnp.float32)
        ea = ea_ref[...]
        t1 = jnp.dot(ea, wef1, preferred_element_type=jnp.float32)
        t2 = jnp.dot(ea, wef2, preferred_element_type=jnp.float32)
        one = jnp.ones((BB, 1), jnp.float32)
        z1 = jnp.zeros((BB, 1), jnp.float32)
        z6 = jnp.zeros((BB, 6), jnp.float32)
        z14 = jnp.zeros((BB, 14), jnp.float32)
        z96 = jnp.zeros((BB, 96), jnp.float32)
        out_ref[...] = jnp.concatenate([one, t1, t2, z6, z1, t2, z14, z96],
                                       axis=1)

    return pl.pallas_call(
        body,
        grid=(E // BB,),
        in_specs=[pl.BlockSpec((BB, 16), lambda i: (i, 0)),
                  pl.BlockSpec((16, 64), lambda i: (0, 0)),
                  pl.BlockSpec((8, 8), lambda i: (0, 0)),
                  pl.BlockSpec((16, 128), lambda i: (0, 0)),
                  pl.BlockSpec((128, 1), lambda i: (0, 0))],
        out_specs=pl.BlockSpec((BB, 128), lambda i: (i, 0)),
        out_shape=jax.ShapeDtypeStruct((E, 128), jnp.float32),
    )(edge_attr, We1, ae1, We2, ae2)


def _mid(acc1, ts1, W2, as2, ad2, b1):
    BN = 1000
    NB = N // BN
    hb = N // (BN * NC)

    def body(acc_ref, ts1_ref, w2_ref, as2_ref, ad2_ref, b1_ref,
             ts2_ref, ad2t_ref, aux_ref):
        a = acc_ref[...]
        deg = jnp.maximum(a[:, 0:1], 1.0)
        ae1l = a[:, 1:9] / deg
        ae2l = a[:, 9:10] / deg
        ts1 = ts1_ref[...]
        asrc1 = ts1[:, 1:9]
        h1 = ts1[:, 16:80]
        adst1 = ts1[:, 113:121]
        wl = jnp.exp(_leaky(asrc1 + adst1 + ae1l))
        denom = a[:, 17:25] + wl + 1e-16
        hr = h1.reshape(BN, 8, 8)
        msg = a[:, 32:96].reshape(BN, 8, 8) + wl[:, :, None] * hr
        o1 = (msg / denom[:, :, None]).reshape(BN, 64) + b1_ref[...]
        hact = jnp.where(o1 > 0, o1, jnp.exp(o1) - 1.0)
        h2 = jnp.dot(hact, w2_ref[...], preferred_element_type=jnp.float32)
        asrc2 = jnp.dot(h2, as2_ref[...], preferred_element_type=jnp.float32)
        adst2 = jnp.dot(h2, ad2_ref[...], preferred_element_type=jnp.float32)
        z1 = jnp.zeros((BN, 1), jnp.float32)
        z14 = jnp.zeros((BN, 14), jnp.float32)
        z112 = jnp.zeros((BN, 112), jnp.float32)
        z126 = jnp.zeros((BN, 126), jnp.float32)
        ts2_ref[...] = jnp.concatenate([z1, asrc2, z14, h2, z112], axis=1)
        ad2t_ref[...] = jnp.concatenate([z1, adst2, z126], axis=1)
        wl2 = jnp.exp(_leaky(asrc2 + adst2 + ae2l))
        aux_ref[...] = jnp.concatenate([wl2, jnp.zeros((BN, 7), jnp.float32)],
                                       axis=1)

    return pl.pallas_call(
        body,
        grid=(NB,),
        in_specs=[pl.BlockSpec((BN, P1), lambda i: (i, 0)),
                  pl.BlockSpec((BN, ROW1), lambda i: (i, 0)),
                  pl.BlockSpec((64, 128), lambda i: (0, 0)),
                  pl.BlockSpec((128, 1), lambda i: (0, 0)),
                  pl.BlockSpec((128, 1), lambda i: (0, 0)),
                  pl.BlockSpec((1, 64), lambda i: (0, 0))],
        out_specs=(pl.BlockSpec((BN, ROW2), lambda i: (i, 0)),
                   pl.BlockSpec((BN, 128), lambda i: (i, 0)),
                   pl.BlockSpec((BN, 8), lambda i: (i, 0))),
        out_shape=(jax.ShapeDtypeStruct((N, ROW2), jnp.float32),
                   jax.ShapeDtypeStruct((N, 128), jnp.float32),
                   jax.ShapeDtypeStruct((N, 8), jnp.float32)),
    )(acc1, ts1, W2, as2, ad2, b1)


def _final(acc2, ts2, aux, b2, batch3d):
    BN = 1000
    NB = N // BN
    hb = N // (BN * NC)

    def body(acc_ref, ts2_ref, aux_ref, b2_ref, bt_ref, out_ref, cnt_ref):
        i = pl.program_id(0)
        a = acc_ref[...]
        wl2 = aux_ref[...][:, 0:1]
        h2 = ts2_ref[...][:, 16:144]
        denom = a[:, 128:129] + wl2 + 1e-16
        o2 = (a[:, 0:128] + wl2 * h2) / denom + b2_ref[...]
        gids = lax.broadcasted_iota(jnp.int32, (G, BN), 0)
        mask = (bt_ref[0] == gids).astype(jnp.float32)
        ssum = jnp.dot(mask, o2, preferred_element_type=jnp.float32)
        cb = jnp.concatenate(
            [mask.sum(axis=1, keepdims=True), jnp.zeros((G, 7), jnp.float32)],
            axis=1)

        @pl.when(i == 0)
        def _():
            out_ref[...] = ssum
            cnt_ref[...] = cb

        @pl.when(i > 0)
        def _():
            out_ref[...] += ssum
            cnt_ref[...] += cb

        @pl.when(i == NB - 1)
        def _():
            out_ref[...] = out_ref[...] / jnp.maximum(cnt_ref[...][:, 0:1], 1.0)

    out, _cnt = pl.pallas_call(
        body,
        grid=(NB,),
        in_specs=[pl.BlockSpec((BN, P2), lambda i: (i, 0)),
                  pl.BlockSpec((BN, ROW2), lambda i: (i, 0)),
                  pl.BlockSpec((BN, 8), lambda i: (i, 0)),
                  pl.BlockSpec((1, 128), lambda i: (0, 0)),
                  pl.BlockSpec((1, 1, BN), lambda i: (i, 0, 0))],
        out_specs=(pl.BlockSpec((G, 128), lambda i: (0, 0)),
                   pl.BlockSpec((G, 8), lambda i: (0, 0))),
        out_shape=(jax.ShapeDtypeStruct((G, 128), jnp.float32),
                   jax.ShapeDtypeStruct((G, 8), jnp.float32)),
    )(acc2, ts2, aux, b2, batch3d)
    return out


def kernel(x, edge_index, edge_attr, batch,
           W1, att_src1, att_dst1, We1, att_e1, b1,
           W2, att_src2, att_dst2, We2, att_e2, b2):
    src = edge_index[0]
    dst = edge_index[1]
    ts1 = _node1(x, W1, att_src1.reshape(8, 8), att_dst1.reshape(8, 8))
    aet = _ae_tab(edge_attr, We1, att_e1.reshape(8, 8), We2,
                  att_e2.reshape(128, 1))
    acc1 = _edge_pass1(ts1, ts1, aet, src, dst)
    ts2, ad2t, aux = _mid(acc1, ts1, W2,
                          att_src2.reshape(128, 1), att_dst2.reshape(128, 1),
                          b1.reshape(1, 64))
    acc2 = _edge_pass2(ts2, ad2t, aet, src, dst)
    return _final(acc2, ts2, aux, b2.reshape(1, 128), batch.reshape(10, 1, 1000))
